# baseline (speedup 1.0000x reference)
import jax
import jax.numpy as jnp
from jax import lax
from jax.experimental import pallas as pl
from jax.experimental.pallas import tpu as pltpu
from jax.experimental.pallas import tpu_sc as plsc

NC = 2
NS = 16
NW = NC * NS
L = 16
CH = 128
U = 8


def _seg_sum_sc(n, g, d):
  assert n % 8 == 0 and d % L == 0 and CH % U == 0
  oct_total = n // 8
  ob, oe = divmod(oct_total, NW)
  max_range = 8 * (ob + 1 if oe else ob)
  nwin = -(-max_range // CH)
  assert nwin >= 2 and max_range >= CH
  nj = d // L
  dw = d + L

  def body(x_hbm, b_hbm, acc_out, xb0, xb1, iv0, iv1, acc_v, sprev, sF, sRow,
           *sems):
    c = lax.axis_index("c")
    s = lax.axis_index("s")
    w = c * NS + s

    a0 = 8 * (w * ob + jnp.minimum(w, oe))
    rng = 8 * (ob + jnp.where(w < oe, 1, 0))

    xbufs = [xb0, xb1]
    ibufs = [iv0, iv1]

    def win_base(k):
      return jnp.minimum(a0 + k * CH, a0 + rng - CH)

    def make_copies(k):
      b = win_base(k)
      cx = pltpu.make_async_copy(x_hbm.at[pl.ds(b, CH)], xbufs[k % 2],
                                 sems[2 * k])
      ci = pltpu.make_async_copy(b_hbm.at[pl.ds(b, CH)], ibufs[k % 2],
                                 sems[2 * k + 1])
      return cx, ci

    copies = [make_copies(k) for k in range(nwin)]
    for k in range(min(2, nwin)):
      copies[k][0].start()
      copies[k][1].start()

    zv = jnp.zeros((L,), jnp.float32)

    def zero_step(i, _):
      for j in range(nj + 1):
        acc_v[i, pl.ds(j * L, L)] = zv
      return 0

    lax.fori_loop(0, g, zero_step, 0)
    for j in range(nj):
      sF[pl.ds(j * L, L)] = zv
    sprev[...] = jnp.full((L,), -1, jnp.int32)
    sRow[...] = jnp.full((L,), 0, jnp.int32) + a0

    lane = lax.iota(jnp.int32, L)
    blk_lane = jnp.minimum(lane, U - 1)

    def flush(prev_v, q, gnow):
      for j in range(nj):
        plsc.store_scatter(acc_v, [prev_v, lane + j * L],
                           q[j] - sF[pl.ds(j * L, L)])
        sF[pl.ds(j * L, L)] = q[j]
      plsc.store_scatter(acc_v, [prev_v, lane + nj * L],
                         (gnow - sRow[...]).astype(jnp.float32))
      sRow[...] = gnow

    def slow_row(xbuf, ibuf, base):
      def step(i, q):
        seg_v = plsc.load_gather(ibuf, [jnp.full((L,), 0, jnp.int32) + i])
        prev_v = sprev[...]

        @pl.when(jnp.any(seg_v != prev_v) & jnp.any(prev_v >= 0))
        def _():
          flush(prev_v, q, jnp.full((L,), 0, jnp.int32) + (base + i))

        sprev[...] = seg_v
        return tuple(q[j] + xbuf[i, pl.ds(j * L, L)] for j in range(nj))

      return step

    def process(k, P):
      xbuf = xbufs[k % 2]
      ibuf = ibufs[k % 2]
      lo = jnp.maximum((k + 1) * CH - rng, 0)
      srow = slow_row(xbuf, ibuf, win_base(k))

      hi0 = jnp.minimum((lo + (U - 1)) // U * U, CH)
      P = lax.fori_loop(lo, hi0, srow, P)

      def blk_step(b, q):
        i0 = b * U
        ids = plsc.load_gather(ibuf, [blk_lane + i0])
        prev_v = sprev[...]

        @pl.when(jnp.any(ids != prev_v))
        def _():
          lax.fori_loop(i0, i0 + U, srow, q)

        for r in range(U):
          q = tuple(q[j] + xbuf[i0 + r, pl.ds(j * L, L)] for j in range(nj))
        return q

      return lax.fori_loop(hi0 // U, CH // U, blk_step, P)

    P = tuple(jnp.zeros((L,), jnp.float32) for _ in range(nj))
    for k in range(nwin):
      copies[k][0].wait()
      copies[k][1].wait()
      P = process(k, P)
      if k + 2 < nwin:
        copies[k + 2][0].start()
        copies[k + 2][1].start()

    prev_v = sprev[...]

    @pl.when(jnp.any(prev_v >= 0))
    def _():
      flush(prev_v, P, jnp.full((L,), 0, jnp.int32) + (a0 + rng))

    pltpu.sync_copy(acc_v, acc_out.at[pl.ds(w * g, g)])

  return pl.kernel(
      body,
      out_type=jax.ShapeDtypeStruct((NW * g, dw), jnp.float32),
      mesh=plsc.VectorSubcoreMesh(core_axis_name="c", subcore_axis_name="s"),
      compiler_params=pltpu.CompilerParams(needs_layout_passes=False),
      scratch_types=[
          pltpu.VMEM((CH, d), jnp.float32),
          pltpu.VMEM((CH, d), jnp.float32),
          pltpu.VMEM((CH,), jnp.int32),
          pltpu.VMEM((CH,), jnp.int32),
          pltpu.VMEM((g, dw), jnp.float32),
          pltpu.VMEM((L,), jnp.int32),
          pltpu.VMEM((d,), jnp.float32),
          pltpu.VMEM((L,), jnp.int32),
      ] + [pltpu.SemaphoreType.DMA] * (2 * nwin),
  )


def _mlp_body(acc_ref, u_ref, w1_ref, b1_ref, w2_ref, b2_ref, o_ref):
  g, du = u_ref.shape
  d = acc_ref.shape[1] - L

  def red_step(i, carry):
    off = pl.multiple_of(i * g, 8)
    return carry + acc_ref[pl.ds(off, g), :]

  sums = lax.fori_loop(1, NW, red_step, acc_ref[pl.ds(0, g), :])

  cnt = sums[:, d:d + 1]
  mean = sums[:, :d] / jnp.maximum(cnt, 1.0)
  pre = (jnp.dot(u_ref[...], w1_ref[pl.ds(0, du), :],
                 preferred_element_type=jnp.float32)
         + jnp.dot(mean, w1_ref[pl.ds(du, d), :],
                   preferred_element_type=jnp.float32)
         + b1_ref[...])
  h = jnp.maximum(pre, 0.0)
  o_ref[...] = (jnp.dot(h, w2_ref[...], preferred_element_type=jnp.float32)
                + b2_ref[...])


@jax.jit
def kernel(x, edge_index, edge_attr, u, batch, W1, b1, W2, b2):
  del edge_index, edge_attr
  n, d = x.shape
  g, du = u.shape

  batch32 = batch.astype(jnp.int32)
  acc = _seg_sum_sc(n, g, d)(x, batch32)

  return pl.pallas_call(
      _mlp_body,
      out_shape=jax.ShapeDtypeStruct((g, W2.shape[1]), jnp.float32),
  )(acc, u, W1, b1.reshape(1, -1), W2, b2.reshape(1, -1))
